# baseline (speedup 1.0000x reference)
import functools

import jax
import jax.numpy as jnp
from jax import lax
from jax.experimental import pallas as pl
from jax.experimental.pallas import tpu as pltpu
from jax.experimental.pallas import tpu_sc as plsc

E = 16
K = 2
DI = 1024
DH = 2048
DO = 1024
N = 8192

GT = 256
FT = 128
NT = N * K // FT + E
M_PAD = NT * FT


def _gating_body(x_ref, wg_ref, meta_ref, cnt_ref, loss_ref, base_ref, imp_ref):
    i = pl.program_id(0)

    @pl.when(i == 0)
    def _init():
        base_ref[...] = jnp.zeros_like(base_ref)
        imp_ref[...] = jnp.zeros_like(imp_ref)

    lg = jnp.dot(x_ref[...], wg_ref[...], preferred_element_type=jnp.float32)
    col = jax.lax.broadcasted_iota(jnp.int32, (GT, E), 1)
    m0 = jnp.max(lg, axis=1, keepdims=True)
    i0 = jnp.min(jnp.where(lg == m0, col, E), axis=1, keepdims=True)
    is0 = col == i0
    lg2 = jnp.where(is0, -jnp.inf, lg)
    m1 = jnp.max(lg2, axis=1, keepdims=True)
    i1 = jnp.min(jnp.where(lg2 == m1, col, E), axis=1, keepdims=True)
    is1 = col == i1
    a = jnp.exp(m1 - m0)
    g0 = 1.0 / (1.0 + a)
    g1 = a / (1.0 + a)
    P0 = is0.astype(jnp.float32)
    P1 = is1.astype(jnp.float32)

    rowi = jax.lax.broadcasted_iota(jnp.int32, (GT, GT), 0)
    colj = jax.lax.broadcasted_iota(jnp.int32, (GT, GT), 1)
    tri = (colj < rowi).astype(jnp.float32)
    cum0 = jnp.dot(tri, P0, preferred_element_type=jnp.float32)
    cum1 = jnp.dot(tri, P1, preferred_element_type=jnp.float32)
    c0 = jnp.sum(P0, axis=0, keepdims=True)
    base = base_ref[...]
    r0 = jnp.sum((cum0 + base) * P0, axis=1, keepdims=True)
    r1 = jnp.sum((cum1 + base + c0) * P1, axis=1, keepdims=True)

    cnt_tile = c0 + jnp.sum(P1, axis=0, keepdims=True)
    imp_tile = jnp.sum(g0 * P0 + g1 * P1, axis=0, keepdims=True)
    base_ref[...] = base + cnt_tile
    imp_ref[...] = imp_ref[...] + imp_tile

    meta_ref[...] = jnp.concatenate(
        [i0.astype(jnp.float32), i1.astype(jnp.float32), r0, r1, g0, g1,
         jnp.zeros((GT, 2), jnp.float32)], axis=1)

    @pl.when(i == pl.num_programs(0) - 1)
    def _fin():
        cnt = base_ref[...]
        imp = imp_ref[...]
        cnt_ref[...] = cnt

        def cv2(v):
            mean = jnp.sum(v) / E
            var = jnp.sum((v - mean) ** 2) / (E - 1)
            return var / (mean * mean + 1e-10)

        loss_ref[...] = (0.01 * (cv2(imp) + cv2(cnt))).reshape(1, 1)


def _gating(x, w_gate):
    return pl.pallas_call(
        _gating_body,
        grid=(N // GT,),
        in_specs=[
            pl.BlockSpec((GT, DI), lambda i: (i, 0)),
            pl.BlockSpec((DI, E), lambda i: (0, 0)),
        ],
        out_specs=[
            pl.BlockSpec((GT, 8), lambda i: (i, 0)),
            pl.BlockSpec((1, E), lambda i: (0, 0)),
            pl.BlockSpec((1, 1), lambda i: (0, 0)),
        ],
        out_shape=[
            jax.ShapeDtypeStruct((N, 8), jnp.float32),
            jax.ShapeDtypeStruct((1, E), jnp.float32),
            jax.ShapeDtypeStruct((1, 1), jnp.float32),
        ],
        scratch_shapes=[
            pltpu.VMEM((1, E), jnp.float32),
            pltpu.VMEM((1, E), jnp.float32),
        ],
    )(x, w_gate)


def _ffn_body(t2e_ref, xs_ref, w1_ref, b1_ref, w2_ref, b2_ref, sg_ref, o_ref):
    i = pl.program_id(0)

    @pl.when(t2e_ref[i] < E)
    def _go():
        h = jnp.dot(xs_ref[...], w1_ref[0], preferred_element_type=jnp.float32)
        h = jnp.maximum(h + b1_ref[0], 0.0)
        lg = jnp.dot(h, w2_ref[0], preferred_element_type=jnp.float32)
        lg = lg + b2_ref[0]
        m = jnp.max(lg, axis=1, keepdims=True)
        p = jnp.exp(lg - m)
        g = sg_ref[0, 0, :].reshape(FT, 1)
        o_ref[...] = p * (g / jnp.sum(p, axis=1, keepdims=True))


def _ffn(xs, W1, b1, W2, b2, sg, t2e):
    def emap(i, t2e_ref):
        return (jnp.minimum(t2e_ref[i], E - 1), 0, 0)

    def emap3(i, t2e_ref):
        return (jnp.minimum(t2e_ref[i], E - 1), 0, 0)

    grid_spec = pltpu.PrefetchScalarGridSpec(
        num_scalar_prefetch=1,
        grid=(NT,),
        in_specs=[
            pl.BlockSpec((FT, DI), lambda i, t: (i, 0)),
            pl.BlockSpec((1, DI, DH), emap),
            pl.BlockSpec((1, 1, DH), emap3),
            pl.BlockSpec((1, DH, DO), emap),
            pl.BlockSpec((1, 1, DO), emap3),
            pl.BlockSpec((1, 1, FT), lambda i, t: (i, 0, 0)),
        ],
        out_specs=pl.BlockSpec((FT, DO), lambda i, t: (i, 0)),
    )
    return pl.pallas_call(
        _ffn_body,
        grid_spec=grid_spec,
        out_shape=jax.ShapeDtypeStruct((M_PAD, DO), jnp.float32),
    )(t2e, xs, W1, b1.reshape(E, 1, DH), W2, b2.reshape(E, 1, DO),
      sg.reshape(NT, 1, FT))


NW = 32
PAIRS = N * K
PPW = PAIRS // NW
CH = 64
NCH = PPW // CH
TPW = N // NW
TCH = CH // K

_SC_MESH = plsc.VectorSubcoreMesh(core_axis_name="c", subcore_axis_name="s")


@functools.partial(
    pl.kernel,
    out_type=[jax.ShapeDtypeStruct((M_PAD, DI), jnp.float32),
              jax.ShapeDtypeStruct((M_PAD,), jnp.float32)],
    mesh=_SC_MESH,
    scratch_types=[
        pltpu.VMEM((CH,), jnp.int32),
        pltpu.VMEM((NCH, CH), jnp.int32),
        pltpu.VMEM((CH,), jnp.float32),
        pltpu.VMEM((CH, DI), jnp.float32),
        pltpu.SemaphoreType.DMA,
        pltpu.SemaphoreType.DMA,
        pltpu.SemaphoreType.DMA,
    ],
)
def _sc_dispatch(x_hbm, tok_hbm, pos_in_hbm, g_hbm,
                 xs_hbm, sg_hbm,
                 tok_v, pos_v, g_v, rows_v, sem_g, sem_s, sem_sg):
    wid = lax.axis_index("s") * 2 + lax.axis_index("c")
    base0 = wid * PPW

    def chunk(c, carry):
        base = base0 + c * CH
        pltpu.sync_copy(tok_hbm.at[pl.ds(base, CH)], tok_v)
        pltpu.sync_copy(pos_in_hbm.at[pl.ds(base, CH)], pos_v.at[c])
        pltpu.sync_copy(g_hbm.at[pl.ds(base, CH)], g_v)
        pltpu.async_copy(x_hbm.at[tok_v], rows_v, sem_g).wait()
        pltpu.async_copy(rows_v, xs_hbm.at[pos_v.at[c]], sem_s).wait()
        pltpu.async_copy(g_v, sg_hbm.at[pos_v.at[c]], sem_sg).wait()
        return carry

    lax.fori_loop(0, NCH, chunk, 0)


@functools.partial(
    pl.kernel,
    out_type=jax.ShapeDtypeStruct((N, DO), jnp.float32),
    mesh=_SC_MESH,
    scratch_types=[
        pltpu.VMEM((CH,), jnp.int32),
        pltpu.VMEM((CH, DO), jnp.float32),
        pltpu.VMEM((TCH, DO), jnp.float32),
        pltpu.SemaphoreType.DMA,
    ],
)
def _sc_combine(o_hbm, pos_hbm,
                y_hbm,
                pos_v, rows_v, y_v, sem):
    wid = lax.axis_index("s") * 2 + lax.axis_index("c")

    def chunk(c, carry):
        pbase = wid * PPW + c * CH
        pltpu.sync_copy(pos_hbm.at[pl.ds(pbase, CH)], pos_v)
        pltpu.async_copy(o_hbm.at[pos_v], rows_v, sem).wait()

        def tok(t, inner):
            for j in range(DO // 16):
                a = rows_v[2 * t, pl.ds(j * 16, 16)]
                b = rows_v[2 * t + 1, pl.ds(j * 16, 16)]
                y_v[t, pl.ds(j * 16, 16)] = a + b
            return inner

        lax.fori_loop(0, TCH, tok, 0)
        pltpu.sync_copy(y_v, y_hbm.at[pl.ds(wid * TPW + c * TCH, TCH)])
        return carry

    lax.fori_loop(0, NCH, chunk, 0)


def kernel(x, w_gate, W1, b1, W2, b2):
    meta, cnt, loss = _gating(x, w_gate)
    e_ids = meta[:, :K].astype(jnp.int32)
    ranks = meta[:, K:2 * K].astype(jnp.int32)
    gates = meta[:, 2 * K:3 * K]
    counts = cnt[0].astype(jnp.int32)

    tile_cnt = (counts + FT - 1) // FT
    tile_off = jnp.cumsum(tile_cnt)
    p_off = jnp.concatenate([jnp.zeros((1,), jnp.int32),
                             tile_off[:-1]]) * FT
    tids = jnp.arange(NT, dtype=jnp.int32)
    t2e = jnp.sum((tids[:, None] >= tile_off[None, :]).astype(jnp.int32), axis=1)

    pos = p_off[e_ids] + ranks
    posf = pos.reshape(-1).astype(jnp.int32)
    tokf = jnp.repeat(jnp.arange(N, dtype=jnp.int32), K)
    xs, sg = _sc_dispatch(x, tokf, posf, gates.reshape(-1))
    o = _ffn(xs, W1, b1, W2, b2, sg, t2e)
    y = _sc_combine(o, posf)
    return (y, loss[0, 0])


# speedup vs baseline: 1.0758x; 1.0758x over previous
import functools

import jax
import jax.numpy as jnp
from jax import lax
from jax.experimental import pallas as pl
from jax.experimental.pallas import tpu as pltpu
from jax.experimental.pallas import tpu_sc as plsc

E = 16
K = 2
DI = 1024
DH = 2048
DO = 1024
N = 8192

GT = 256
FT = 512
NT = N * K // FT + E
M_PAD = NT * FT


def _gating_body(x_ref, wg_ref, meta_ref, cnt_ref, loss_ref, base_ref, imp_ref):
    i = pl.program_id(0)

    @pl.when(i == 0)
    def _init():
        base_ref[...] = jnp.zeros_like(base_ref)
        imp_ref[...] = jnp.zeros_like(imp_ref)

    lg = jnp.dot(x_ref[...], wg_ref[...], preferred_element_type=jnp.float32)
    col = jax.lax.broadcasted_iota(jnp.int32, (GT, E), 1)
    m0 = jnp.max(lg, axis=1, keepdims=True)
    i0 = jnp.min(jnp.where(lg == m0, col, E), axis=1, keepdims=True)
    is0 = col == i0
    lg2 = jnp.where(is0, -jnp.inf, lg)
    m1 = jnp.max(lg2, axis=1, keepdims=True)
    i1 = jnp.min(jnp.where(lg2 == m1, col, E), axis=1, keepdims=True)
    is1 = col == i1
    a = jnp.exp(m1 - m0)
    g0 = 1.0 / (1.0 + a)
    g1 = a / (1.0 + a)
    P0 = is0.astype(jnp.float32)
    P1 = is1.astype(jnp.float32)

    rowi = jax.lax.broadcasted_iota(jnp.int32, (GT, GT), 0)
    colj = jax.lax.broadcasted_iota(jnp.int32, (GT, GT), 1)
    tri = (colj < rowi).astype(jnp.float32)
    cum0 = jnp.dot(tri, P0, preferred_element_type=jnp.float32)
    cum1 = jnp.dot(tri, P1, preferred_element_type=jnp.float32)
    c0 = jnp.sum(P0, axis=0, keepdims=True)
    base = base_ref[...]
    r0 = jnp.sum((cum0 + base) * P0, axis=1, keepdims=True)
    r1 = jnp.sum((cum1 + base + c0) * P1, axis=1, keepdims=True)

    cnt_tile = c0 + jnp.sum(P1, axis=0, keepdims=True)
    imp_tile = jnp.sum(g0 * P0 + g1 * P1, axis=0, keepdims=True)
    base_ref[...] = base + cnt_tile
    imp_ref[...] = imp_ref[...] + imp_tile

    meta_ref[...] = jnp.concatenate(
        [i0.astype(jnp.float32), i1.astype(jnp.float32), r0, r1, g0, g1,
         jnp.zeros((GT, 2), jnp.float32)], axis=1)

    @pl.when(i == pl.num_programs(0) - 1)
    def _fin():
        cnt = base_ref[...]
        imp = imp_ref[...]
        cnt_ref[...] = cnt

        def cv2(v):
            mean = jnp.sum(v) / E
            var = jnp.sum((v - mean) ** 2) / (E - 1)
            return var / (mean * mean + 1e-10)

        loss_ref[...] = (0.01 * (cv2(imp) + cv2(cnt))).reshape(1, 1)


def _gating(x, w_gate):
    return pl.pallas_call(
        _gating_body,
        grid=(N // GT,),
        in_specs=[
            pl.BlockSpec((GT, DI), lambda i: (i, 0)),
            pl.BlockSpec((DI, E), lambda i: (0, 0)),
        ],
        out_specs=[
            pl.BlockSpec((GT, 8), lambda i: (i, 0)),
            pl.BlockSpec((1, E), lambda i: (0, 0)),
            pl.BlockSpec((1, 1), lambda i: (0, 0)),
        ],
        out_shape=[
            jax.ShapeDtypeStruct((N, 8), jnp.float32),
            jax.ShapeDtypeStruct((1, E), jnp.float32),
            jax.ShapeDtypeStruct((1, 1), jnp.float32),
        ],
        scratch_shapes=[
            pltpu.VMEM((1, E), jnp.float32),
            pltpu.VMEM((1, E), jnp.float32),
        ],
    )(x, w_gate)


def _ffn_body(t2e_ref, xs_ref, w1_ref, b1_ref, w2_ref, b2_ref, sg_ref, o_ref):
    i = pl.program_id(0)

    @pl.when(t2e_ref[i] < E)
    def _go():
        h = jnp.dot(xs_ref[...], w1_ref[0], preferred_element_type=jnp.float32)
        h = jnp.maximum(h + b1_ref[0], 0.0)
        lg = jnp.dot(h, w2_ref[0], preferred_element_type=jnp.float32)
        lg = lg + b2_ref[0]
        m = jnp.max(lg, axis=1, keepdims=True)
        p = jnp.exp(lg - m)
        g = sg_ref[0, 0, :].reshape(FT, 1)
        o_ref[...] = p * (g / jnp.sum(p, axis=1, keepdims=True))


def _ffn(xs, W1, b1, W2, b2, sg, t2e):
    def emap(i, t2e_ref):
        return (jnp.minimum(t2e_ref[i], E - 1), 0, 0)

    def emap3(i, t2e_ref):
        return (jnp.minimum(t2e_ref[i], E - 1), 0, 0)

    grid_spec = pltpu.PrefetchScalarGridSpec(
        num_scalar_prefetch=1,
        grid=(NT,),
        in_specs=[
            pl.BlockSpec((FT, DI), lambda i, t: (i, 0)),
            pl.BlockSpec((1, DI, DH), emap),
            pl.BlockSpec((1, 1, DH), emap3),
            pl.BlockSpec((1, DH, DO), emap),
            pl.BlockSpec((1, 1, DO), emap3),
            pl.BlockSpec((1, 1, FT), lambda i, t: (i, 0, 0)),
        ],
        out_specs=pl.BlockSpec((FT, DO), lambda i, t: (i, 0)),
    )
    return pl.pallas_call(
        _ffn_body,
        grid_spec=grid_spec,
        out_shape=jax.ShapeDtypeStruct((M_PAD, DO), jnp.float32),
    )(t2e, xs, W1, b1.reshape(E, 1, DH), W2, b2.reshape(E, 1, DO),
      sg.reshape(NT, 1, FT))


NW = 32
PAIRS = N * K
PPW = PAIRS // NW
CH = 64
NCH = PPW // CH
TPW = N // NW
TCH = CH // K

_SC_MESH = plsc.VectorSubcoreMesh(core_axis_name="c", subcore_axis_name="s")


@functools.partial(
    pl.kernel,
    out_type=[jax.ShapeDtypeStruct((M_PAD, DI), jnp.float32),
              jax.ShapeDtypeStruct((M_PAD,), jnp.float32)],
    mesh=_SC_MESH,
    scratch_types=[
        pltpu.VMEM((CH,), jnp.int32),
        pltpu.VMEM((NCH, CH), jnp.int32),
        pltpu.VMEM((CH,), jnp.float32),
        pltpu.VMEM((CH, DI), jnp.float32),
        pltpu.SemaphoreType.DMA,
        pltpu.SemaphoreType.DMA,
        pltpu.SemaphoreType.DMA,
    ],
)
def _sc_dispatch(x_hbm, tok_hbm, pos_in_hbm, g_hbm,
                 xs_hbm, sg_hbm,
                 tok_v, pos_v, g_v, rows_v, sem_g, sem_s, sem_sg):
    wid = lax.axis_index("s") * 2 + lax.axis_index("c")
    base0 = wid * PPW

    def chunk(c, carry):
        base = base0 + c * CH
        pltpu.sync_copy(tok_hbm.at[pl.ds(base, CH)], tok_v)
        pltpu.sync_copy(pos_in_hbm.at[pl.ds(base, CH)], pos_v.at[c])
        pltpu.sync_copy(g_hbm.at[pl.ds(base, CH)], g_v)
        pltpu.async_copy(x_hbm.at[tok_v], rows_v, sem_g).wait()
        pltpu.async_copy(rows_v, xs_hbm.at[pos_v.at[c]], sem_s).wait()
        pltpu.async_copy(g_v, sg_hbm.at[pos_v.at[c]], sem_sg).wait()
        return carry

    lax.fori_loop(0, NCH, chunk, 0)


@functools.partial(
    pl.kernel,
    out_type=jax.ShapeDtypeStruct((N, DO), jnp.float32),
    mesh=_SC_MESH,
    scratch_types=[
        pltpu.VMEM((CH,), jnp.int32),
        pltpu.VMEM((CH, DO), jnp.float32),
        pltpu.VMEM((TCH, DO), jnp.float32),
        pltpu.SemaphoreType.DMA,
    ],
)
def _sc_combine(o_hbm, pos_hbm,
                y_hbm,
                pos_v, rows_v, y_v, sem):
    wid = lax.axis_index("s") * 2 + lax.axis_index("c")

    def chunk(c, carry):
        pbase = wid * PPW + c * CH
        pltpu.sync_copy(pos_hbm.at[pl.ds(pbase, CH)], pos_v)
        pltpu.async_copy(o_hbm.at[pos_v], rows_v, sem).wait()

        def tok(t, inner):
            for j in range(DO // 16):
                a = rows_v[2 * t, pl.ds(j * 16, 16)]
                b = rows_v[2 * t + 1, pl.ds(j * 16, 16)]
                y_v[t, pl.ds(j * 16, 16)] = a + b
            return inner

        lax.fori_loop(0, TCH, tok, 0)
        pltpu.sync_copy(y_v, y_hbm.at[pl.ds(wid * TPW + c * TCH, TCH)])
        return carry

    lax.fori_loop(0, NCH, chunk, 0)


def kernel(x, w_gate, W1, b1, W2, b2):
    meta, cnt, loss = _gating(x, w_gate)
    e_ids = meta[:, :K].astype(jnp.int32)
    ranks = meta[:, K:2 * K].astype(jnp.int32)
    gates = meta[:, 2 * K:3 * K]
    counts = cnt[0].astype(jnp.int32)

    tile_cnt = (counts + FT - 1) // FT
    tile_off = jnp.cumsum(tile_cnt)
    p_off = jnp.concatenate([jnp.zeros((1,), jnp.int32),
                             tile_off[:-1]]) * FT
    tids = jnp.arange(NT, dtype=jnp.int32)
    t2e = jnp.sum((tids[:, None] >= tile_off[None, :]).astype(jnp.int32), axis=1)

    pos = p_off[e_ids] + ranks
    posf = pos.reshape(-1).astype(jnp.int32)
    tokf = jnp.repeat(jnp.arange(N, dtype=jnp.int32), K)
    xs, sg = _sc_dispatch(x, tokf, posf, gates.reshape(-1))
    o = _ffn(xs, W1, b1, W2, b2, sg, t2e)
    y = _sc_combine(o, posf)
    return (y, loss[0, 0])


# speedup vs baseline: 1.0764x; 1.0006x over previous
import functools

import jax
import jax.numpy as jnp
from jax import lax
from jax.experimental import pallas as pl
from jax.experimental.pallas import tpu as pltpu
from jax.experimental.pallas import tpu_sc as plsc

E = 16
K = 2
DI = 1024
DH = 2048
DO = 1024
N = 8192

GT = 256
FT = 512
NT = N * K // FT + E
M_PAD = NT * FT


def _gating_body(x_ref, wg_ref, meta_ref, cnt_ref, loss_ref, base_ref, imp_ref):
    i = pl.program_id(0)

    @pl.when(i == 0)
    def _init():
        base_ref[...] = jnp.zeros_like(base_ref)
        imp_ref[...] = jnp.zeros_like(imp_ref)

    lg = jnp.dot(x_ref[...], wg_ref[...], preferred_element_type=jnp.float32)
    col = jax.lax.broadcasted_iota(jnp.int32, (GT, E), 1)
    m0 = jnp.max(lg, axis=1, keepdims=True)
    i0 = jnp.min(jnp.where(lg == m0, col, E), axis=1, keepdims=True)
    is0 = col == i0
    lg2 = jnp.where(is0, -jnp.inf, lg)
    m1 = jnp.max(lg2, axis=1, keepdims=True)
    i1 = jnp.min(jnp.where(lg2 == m1, col, E), axis=1, keepdims=True)
    is1 = col == i1
    a = jnp.exp(m1 - m0)
    g0 = 1.0 / (1.0 + a)
    g1 = a / (1.0 + a)
    P0 = is0.astype(jnp.float32)
    P1 = is1.astype(jnp.float32)

    rowi = jax.lax.broadcasted_iota(jnp.int32, (GT, GT), 0)
    colj = jax.lax.broadcasted_iota(jnp.int32, (GT, GT), 1)
    tri = (colj < rowi).astype(jnp.float32)
    cum0 = jnp.dot(tri, P0, preferred_element_type=jnp.float32)
    cum1 = jnp.dot(tri, P1, preferred_element_type=jnp.float32)
    c0 = jnp.sum(P0, axis=0, keepdims=True)
    base = base_ref[...]
    r0 = jnp.sum((cum0 + base) * P0, axis=1, keepdims=True)
    r1 = jnp.sum((cum1 + base + c0) * P1, axis=1, keepdims=True)

    cnt_tile = c0 + jnp.sum(P1, axis=0, keepdims=True)
    imp_tile = jnp.sum(g0 * P0 + g1 * P1, axis=0, keepdims=True)
    base_ref[...] = base + cnt_tile
    imp_ref[...] = imp_ref[...] + imp_tile

    meta_ref[...] = jnp.concatenate(
        [i0.astype(jnp.float32), i1.astype(jnp.float32), r0, r1, g0, g1,
         jnp.zeros((GT, 2), jnp.float32)], axis=1)

    @pl.when(i == pl.num_programs(0) - 1)
    def _fin():
        cnt = base_ref[...]
        imp = imp_ref[...]
        cnt_ref[...] = cnt

        def cv2(v):
            mean = jnp.sum(v) / E
            var = jnp.sum((v - mean) ** 2) / (E - 1)
            return var / (mean * mean + 1e-10)

        loss_ref[...] = (0.01 * (cv2(imp) + cv2(cnt))).reshape(1, 1)


def _gating(x, w_gate):
    return pl.pallas_call(
        _gating_body,
        grid=(N // GT,),
        in_specs=[
            pl.BlockSpec((GT, DI), lambda i: (i, 0)),
            pl.BlockSpec((DI, E), lambda i: (0, 0)),
        ],
        out_specs=[
            pl.BlockSpec((GT, 8), lambda i: (i, 0)),
            pl.BlockSpec((1, E), lambda i: (0, 0)),
            pl.BlockSpec((1, 1), lambda i: (0, 0)),
        ],
        out_shape=[
            jax.ShapeDtypeStruct((N, 8), jnp.float32),
            jax.ShapeDtypeStruct((1, E), jnp.float32),
            jax.ShapeDtypeStruct((1, 1), jnp.float32),
        ],
        scratch_shapes=[
            pltpu.VMEM((1, E), jnp.float32),
            pltpu.VMEM((1, E), jnp.float32),
        ],
    )(x, w_gate)


def _ffn_body(t2e_ref, xs_ref, w1_ref, b1_ref, w2_ref, b2_ref, sg_ref, o_ref):
    i = pl.program_id(0)

    @pl.when(t2e_ref[i] < E)
    def _go():
        h = jnp.dot(xs_ref[...].astype(jnp.bfloat16),
                    w1_ref[0].astype(jnp.bfloat16),
                    preferred_element_type=jnp.float32)
        h = jnp.maximum(h + b1_ref[0], 0.0)
        lg = jnp.dot(h.astype(jnp.bfloat16),
                     w2_ref[0].astype(jnp.bfloat16),
                     preferred_element_type=jnp.float32)
        lg = lg + b2_ref[0]
        m = jnp.max(lg, axis=1, keepdims=True)
        p = jnp.exp(lg - m)
        g = sg_ref[0, 0, :].reshape(FT, 1)
        o_ref[...] = p * (g / jnp.sum(p, axis=1, keepdims=True))


def _ffn(xs, W1, b1, W2, b2, sg, t2e):
    def emap(i, t2e_ref):
        return (jnp.minimum(t2e_ref[i], E - 1), 0, 0)

    def emap3(i, t2e_ref):
        return (jnp.minimum(t2e_ref[i], E - 1), 0, 0)

    grid_spec = pltpu.PrefetchScalarGridSpec(
        num_scalar_prefetch=1,
        grid=(NT,),
        in_specs=[
            pl.BlockSpec((FT, DI), lambda i, t: (i, 0)),
            pl.BlockSpec((1, DI, DH), emap),
            pl.BlockSpec((1, 1, DH), emap3),
            pl.BlockSpec((1, DH, DO), emap),
            pl.BlockSpec((1, 1, DO), emap3),
            pl.BlockSpec((1, 1, FT), lambda i, t: (i, 0, 0)),
        ],
        out_specs=pl.BlockSpec((FT, DO), lambda i, t: (i, 0)),
    )
    return pl.pallas_call(
        _ffn_body,
        grid_spec=grid_spec,
        out_shape=jax.ShapeDtypeStruct((M_PAD, DO), jnp.float32),
    )(t2e, xs, W1, b1.reshape(E, 1, DH), W2, b2.reshape(E, 1, DO),
      sg.reshape(NT, 1, FT))


NW = 32
PAIRS = N * K
PPW = PAIRS // NW
CH = 64
NCH = PPW // CH
TPW = N // NW
TCH = CH // K

_SC_MESH = plsc.VectorSubcoreMesh(core_axis_name="c", subcore_axis_name="s")


@functools.partial(
    pl.kernel,
    out_type=[jax.ShapeDtypeStruct((M_PAD, DI), jnp.float32),
              jax.ShapeDtypeStruct((M_PAD,), jnp.float32)],
    mesh=_SC_MESH,
    scratch_types=[
        pltpu.VMEM((CH,), jnp.int32),
        pltpu.VMEM((NCH, CH), jnp.int32),
        pltpu.VMEM((CH,), jnp.float32),
        pltpu.VMEM((CH, DI), jnp.float32),
        pltpu.SemaphoreType.DMA,
        pltpu.SemaphoreType.DMA,
        pltpu.SemaphoreType.DMA,
    ],
)
def _sc_dispatch(x_hbm, tok_hbm, pos_in_hbm, g_hbm,
                 xs_hbm, sg_hbm,
                 tok_v, pos_v, g_v, rows_v, sem_g, sem_s, sem_sg):
    wid = lax.axis_index("s") * 2 + lax.axis_index("c")
    base0 = wid * PPW

    def chunk(c, carry):
        base = base0 + c * CH
        pltpu.sync_copy(tok_hbm.at[pl.ds(base, CH)], tok_v)
        pltpu.sync_copy(pos_in_hbm.at[pl.ds(base, CH)], pos_v.at[c])
        pltpu.sync_copy(g_hbm.at[pl.ds(base, CH)], g_v)
        pltpu.async_copy(x_hbm.at[tok_v], rows_v, sem_g).wait()
        pltpu.async_copy(rows_v, xs_hbm.at[pos_v.at[c]], sem_s).wait()
        pltpu.async_copy(g_v, sg_hbm.at[pos_v.at[c]], sem_sg).wait()
        return carry

    lax.fori_loop(0, NCH, chunk, 0)


@functools.partial(
    pl.kernel,
    out_type=jax.ShapeDtypeStruct((N, DO), jnp.float32),
    mesh=_SC_MESH,
    scratch_types=[
        pltpu.VMEM((CH,), jnp.int32),
        pltpu.VMEM((CH, DO), jnp.float32),
        pltpu.VMEM((TCH, DO), jnp.float32),
        pltpu.SemaphoreType.DMA,
    ],
)
def _sc_combine(o_hbm, pos_hbm,
                y_hbm,
                pos_v, rows_v, y_v, sem):
    wid = lax.axis_index("s") * 2 + lax.axis_index("c")

    def chunk(c, carry):
        pbase = wid * PPW + c * CH
        pltpu.sync_copy(pos_hbm.at[pl.ds(pbase, CH)], pos_v)
        pltpu.async_copy(o_hbm.at[pos_v], rows_v, sem).wait()

        def tok(t, inner):
            for j in range(DO // 16):
                a = rows_v[2 * t, pl.ds(j * 16, 16)]
                b = rows_v[2 * t + 1, pl.ds(j * 16, 16)]
                y_v[t, pl.ds(j * 16, 16)] = a + b
            return inner

        lax.fori_loop(0, TCH, tok, 0)
        pltpu.sync_copy(y_v, y_hbm.at[pl.ds(wid * TPW + c * TCH, TCH)])
        return carry

    lax.fori_loop(0, NCH, chunk, 0)


def kernel(x, w_gate, W1, b1, W2, b2):
    meta, cnt, loss = _gating(x, w_gate)
    e_ids = meta[:, :K].astype(jnp.int32)
    ranks = meta[:, K:2 * K].astype(jnp.int32)
    gates = meta[:, 2 * K:3 * K]
    counts = cnt[0].astype(jnp.int32)

    tile_cnt = (counts + FT - 1) // FT
    tile_off = jnp.cumsum(tile_cnt)
    p_off = jnp.concatenate([jnp.zeros((1,), jnp.int32),
                             tile_off[:-1]]) * FT
    tids = jnp.arange(NT, dtype=jnp.int32)
    t2e = jnp.sum((tids[:, None] >= tile_off[None, :]).astype(jnp.int32), axis=1)

    pos = p_off[e_ids] + ranks
    posf = pos.reshape(-1).astype(jnp.int32)
    tokf = jnp.repeat(jnp.arange(N, dtype=jnp.int32), K)
    xs, sg = _sc_dispatch(x, tokf, posf, gates.reshape(-1))
    o = _ffn(xs, W1, b1, W2, b2, sg, t2e)
    y = _sc_combine(o, posf)
    return (y, loss[0, 0])


# speedup vs baseline: 1.1227x; 1.0431x over previous
import functools

import jax
import jax.numpy as jnp
from jax import lax
from jax.experimental import pallas as pl
from jax.experimental.pallas import tpu as pltpu
from jax.experimental.pallas import tpu_sc as plsc

E = 16
K = 2
DI = 1024
DH = 2048
DO = 1024
N = 8192

GT = 256
FT = 512
NT = N * K // FT + E
M_PAD = NT * FT


def _gating_body(x_ref, wg_ref, meta_ref, cnt_ref, loss_ref, base_ref, imp_ref):
    i = pl.program_id(0)

    @pl.when(i == 0)
    def _init():
        base_ref[...] = jnp.zeros_like(base_ref)
        imp_ref[...] = jnp.zeros_like(imp_ref)

    lg = jnp.dot(x_ref[...], wg_ref[...], preferred_element_type=jnp.float32)
    col = jax.lax.broadcasted_iota(jnp.int32, (GT, E), 1)
    m0 = jnp.max(lg, axis=1, keepdims=True)
    i0 = jnp.min(jnp.where(lg == m0, col, E), axis=1, keepdims=True)
    is0 = col == i0
    lg2 = jnp.where(is0, -jnp.inf, lg)
    m1 = jnp.max(lg2, axis=1, keepdims=True)
    i1 = jnp.min(jnp.where(lg2 == m1, col, E), axis=1, keepdims=True)
    is1 = col == i1
    a = jnp.exp(m1 - m0)
    g0 = 1.0 / (1.0 + a)
    g1 = a / (1.0 + a)
    P0 = is0.astype(jnp.float32)
    P1 = is1.astype(jnp.float32)

    rowi = jax.lax.broadcasted_iota(jnp.int32, (GT, GT), 0)
    colj = jax.lax.broadcasted_iota(jnp.int32, (GT, GT), 1)
    tri = (colj < rowi).astype(jnp.float32)
    cum0 = jnp.dot(tri, P0, preferred_element_type=jnp.float32)
    cum1 = jnp.dot(tri, P1, preferred_element_type=jnp.float32)
    c0 = jnp.sum(P0, axis=0, keepdims=True)
    base = base_ref[...]
    r0 = jnp.sum((cum0 + base) * P0, axis=1, keepdims=True)
    r1 = jnp.sum((cum1 + base + c0) * P1, axis=1, keepdims=True)

    cnt_tile = c0 + jnp.sum(P1, axis=0, keepdims=True)
    imp_tile = jnp.sum(g0 * P0 + g1 * P1, axis=0, keepdims=True)
    base_ref[...] = base + cnt_tile
    imp_ref[...] = imp_ref[...] + imp_tile

    meta_ref[...] = jnp.concatenate(
        [i0.astype(jnp.float32), i1.astype(jnp.float32), r0, r1, g0, g1,
         jnp.zeros((GT, 2), jnp.float32)], axis=1)

    @pl.when(i == pl.num_programs(0) - 1)
    def _fin():
        cnt = base_ref[...]
        imp = imp_ref[...]
        cnt_ref[...] = cnt

        def cv2(v):
            mean = jnp.sum(v) / E
            var = jnp.sum((v - mean) ** 2) / (E - 1)
            return var / (mean * mean + 1e-10)

        loss_ref[...] = (0.01 * (cv2(imp) + cv2(cnt))).reshape(1, 1)


def _gating(x, w_gate):
    return pl.pallas_call(
        _gating_body,
        grid=(N // GT,),
        in_specs=[
            pl.BlockSpec((GT, DI), lambda i: (i, 0)),
            pl.BlockSpec((DI, E), lambda i: (0, 0)),
        ],
        out_specs=[
            pl.BlockSpec((GT, 8), lambda i: (i, 0)),
            pl.BlockSpec((1, E), lambda i: (0, 0)),
            pl.BlockSpec((1, 1), lambda i: (0, 0)),
        ],
        out_shape=[
            jax.ShapeDtypeStruct((N, 8), jnp.float32),
            jax.ShapeDtypeStruct((1, E), jnp.float32),
            jax.ShapeDtypeStruct((1, 1), jnp.float32),
        ],
        scratch_shapes=[
            pltpu.VMEM((1, E), jnp.float32),
            pltpu.VMEM((1, E), jnp.float32),
        ],
    )(x, w_gate)


def _ffn_body(t2e_ref, xs_ref, w1_ref, b1_ref, w2_ref, b2_ref, sg_ref, o_ref):
    i = pl.program_id(0)

    @pl.when(t2e_ref[i] < E)
    def _go():
        h = jnp.dot(xs_ref[...].astype(jnp.bfloat16),
                    w1_ref[0].astype(jnp.bfloat16),
                    preferred_element_type=jnp.float32)
        h = jnp.maximum(h + b1_ref[0], 0.0)
        lg = jnp.dot(h.astype(jnp.bfloat16),
                     w2_ref[0].astype(jnp.bfloat16),
                     preferred_element_type=jnp.float32)
        lg = lg + b2_ref[0]
        m = jnp.max(lg, axis=1, keepdims=True)
        p = jnp.exp(lg - m)
        g = sg_ref[0, 0, :].reshape(FT, 1)
        o_ref[...] = p * (g / jnp.sum(p, axis=1, keepdims=True))


def _ffn(xs, W1, b1, W2, b2, sg, t2e):
    def emap(i, t2e_ref):
        return (jnp.minimum(t2e_ref[i], E - 1), 0, 0)

    def emap3(i, t2e_ref):
        return (jnp.minimum(t2e_ref[i], E - 1), 0, 0)

    grid_spec = pltpu.PrefetchScalarGridSpec(
        num_scalar_prefetch=1,
        grid=(NT,),
        in_specs=[
            pl.BlockSpec((FT, DI), lambda i, t: (i, 0)),
            pl.BlockSpec((1, DI, DH), emap),
            pl.BlockSpec((1, 1, DH), emap3),
            pl.BlockSpec((1, DH, DO), emap),
            pl.BlockSpec((1, 1, DO), emap3),
            pl.BlockSpec((1, 1, FT), lambda i, t: (i, 0, 0)),
        ],
        out_specs=pl.BlockSpec((FT, DO), lambda i, t: (i, 0)),
    )
    return pl.pallas_call(
        _ffn_body,
        grid_spec=grid_spec,
        out_shape=jax.ShapeDtypeStruct((M_PAD, DO), jnp.float32),
    )(t2e, xs, W1, b1.reshape(E, 1, DH), W2, b2.reshape(E, 1, DO),
      sg.reshape(NT, 1, FT))


NW = 32
PAIRS = N * K
PPW = PAIRS // NW
CH = 32
NCH = PPW // CH
NCH2 = NCH // 2
TPW = N // NW
TCH = CH // K

_SC_MESH = plsc.VectorSubcoreMesh(core_axis_name="c", subcore_axis_name="s")


@functools.partial(
    pl.kernel,
    out_type=[jax.ShapeDtypeStruct((M_PAD, DI), jnp.float32),
              jax.ShapeDtypeStruct((M_PAD,), jnp.float32)],
    mesh=_SC_MESH,
    scratch_types=[
        pltpu.VMEM((2, CH), jnp.int32),
        pltpu.VMEM((NCH, CH), jnp.int32),
        pltpu.VMEM((2, CH), jnp.float32),
        pltpu.VMEM((CH, DI), jnp.float32),
        pltpu.VMEM((CH, DI), jnp.float32),
        pltpu.SemaphoreType.DMA,
        pltpu.SemaphoreType.DMA,
        pltpu.SemaphoreType.DMA,
        pltpu.SemaphoreType.DMA,
        pltpu.SemaphoreType.DMA,
        pltpu.SemaphoreType.DMA,
    ],
)
def _sc_dispatch(x_hbm, tok_hbm, pos_in_hbm, g_hbm,
                 xs_hbm, sg_hbm,
                 tok_v, pos_v, g_v, rows0, rows1,
                 gsem0, gsem1, ssem0, ssem1, tsem0, tsem1):
    wid = lax.axis_index("s") * 2 + lax.axis_index("c")
    base0 = wid * PPW

    def stage(c, b):
        base = base0 + c * CH
        pltpu.sync_copy(tok_hbm.at[pl.ds(base, CH)], tok_v.at[b])
        pltpu.sync_copy(pos_in_hbm.at[pl.ds(base, CH)], pos_v.at[c])
        pltpu.sync_copy(g_hbm.at[pl.ds(base, CH)], g_v.at[b])

    stage(0, 0)
    pltpu.async_copy(x_hbm.at[tok_v.at[0]], rows0, gsem0)

    def body(c2, carry):
        c0 = 2 * c2
        c1 = c0 + 1

        @pl.when(c2 >= 1)
        def _wait_b1():
            pltpu.make_async_copy(rows1, xs_hbm.at[pos_v.at[c1 - 2]], ssem1).wait()
            pltpu.make_async_copy(g_v.at[1], sg_hbm.at[pos_v.at[c1 - 2]], tsem1).wait()

        stage(c1, 1)
        pltpu.async_copy(x_hbm.at[tok_v.at[1]], rows1, gsem1)

        pltpu.make_async_copy(x_hbm.at[tok_v.at[0]], rows0, gsem0).wait()
        pltpu.async_copy(rows0, xs_hbm.at[pos_v.at[c0]], ssem0)
        pltpu.async_copy(g_v.at[0], sg_hbm.at[pos_v.at[c0]], tsem0)

        pltpu.make_async_copy(rows0, xs_hbm.at[pos_v.at[c0]], ssem0).wait()
        pltpu.make_async_copy(g_v.at[0], sg_hbm.at[pos_v.at[c0]], tsem0).wait()
        cn = jnp.minimum(c0 + 2, NCH - 1)
        stage(cn, 0)
        pltpu.async_copy(x_hbm.at[tok_v.at[0]], rows0, gsem0)

        pltpu.make_async_copy(x_hbm.at[tok_v.at[1]], rows1, gsem1).wait()
        pltpu.async_copy(rows1, xs_hbm.at[pos_v.at[c1]], ssem1)
        pltpu.async_copy(g_v.at[1], sg_hbm.at[pos_v.at[c1]], tsem1)
        return carry

    lax.fori_loop(0, NCH2, body, 0)
    pltpu.make_async_copy(x_hbm.at[tok_v.at[0]], rows0, gsem0).wait()
    pltpu.make_async_copy(rows1, xs_hbm.at[pos_v.at[NCH - 1]], ssem1).wait()
    pltpu.make_async_copy(g_v.at[1], sg_hbm.at[pos_v.at[NCH - 1]], tsem1).wait()


@functools.partial(
    pl.kernel,
    out_type=jax.ShapeDtypeStruct((N, DO), jnp.float32),
    mesh=_SC_MESH,
    scratch_types=[
        pltpu.VMEM((2, CH), jnp.int32),
        pltpu.VMEM((CH, DO), jnp.float32),
        pltpu.VMEM((CH, DO), jnp.float32),
        pltpu.VMEM((TCH, DO), jnp.float32),
        pltpu.SemaphoreType.DMA,
        pltpu.SemaphoreType.DMA,
    ],
)
def _sc_combine(o_hbm, pos_hbm,
                y_hbm,
                pos_v, rows0, rows1, y_v, sem0, sem1):
    wid = lax.axis_index("s") * 2 + lax.axis_index("c")

    def stage_fire(c, b, rows, sem):
        pltpu.sync_copy(pos_hbm.at[pl.ds(wid * PPW + c * CH, CH)], pos_v.at[b])
        pltpu.async_copy(o_hbm.at[pos_v.at[b]], rows, sem)

    def compute(c, rows):
        def tok(t, inner):
            for j in range(DO // 16):
                a = rows[2 * t, pl.ds(j * 16, 16)]
                b = rows[2 * t + 1, pl.ds(j * 16, 16)]
                y_v[t, pl.ds(j * 16, 16)] = a + b
            return inner

        lax.fori_loop(0, TCH, tok, 0)
        pltpu.sync_copy(y_v, y_hbm.at[pl.ds(wid * TPW + c * TCH, TCH)])

    stage_fire(0, 0, rows0, sem0)

    def body(c2, carry):
        c0 = 2 * c2
        c1 = c0 + 1
        stage_fire(c1, 1, rows1, sem1)
        pltpu.make_async_copy(o_hbm.at[pos_v.at[0]], rows0, sem0).wait()
        compute(c0, rows0)
        stage_fire(jnp.minimum(c0 + 2, NCH - 1), 0, rows0, sem0)
        pltpu.make_async_copy(o_hbm.at[pos_v.at[1]], rows1, sem1).wait()
        compute(c1, rows1)
        return carry

    lax.fori_loop(0, NCH2, body, 0)
    pltpu.make_async_copy(o_hbm.at[pos_v.at[0]], rows0, sem0).wait()


def kernel(x, w_gate, W1, b1, W2, b2):
    meta, cnt, loss = _gating(x, w_gate)
    e_ids = meta[:, :K].astype(jnp.int32)
    ranks = meta[:, K:2 * K].astype(jnp.int32)
    gates = meta[:, 2 * K:3 * K]
    counts = cnt[0].astype(jnp.int32)

    tile_cnt = (counts + FT - 1) // FT
    tile_off = jnp.cumsum(tile_cnt)
    p_off = jnp.concatenate([jnp.zeros((1,), jnp.int32),
                             tile_off[:-1]]) * FT
    tids = jnp.arange(NT, dtype=jnp.int32)
    t2e = jnp.sum((tids[:, None] >= tile_off[None, :]).astype(jnp.int32), axis=1)

    pos = p_off[e_ids] + ranks
    posf = pos.reshape(-1).astype(jnp.int32)
    tokf = jnp.repeat(jnp.arange(N, dtype=jnp.int32), K)
    xs, sg = _sc_dispatch(x, tokf, posf, gates.reshape(-1))
    o = _ffn(xs, W1, b1, W2, b2, sg, t2e)
    y = _sc_combine(o, posf)
    return (y, loss[0, 0])


# speedup vs baseline: 1.2824x; 1.1422x over previous
import functools

import jax
import jax.numpy as jnp
from jax import lax
from jax.experimental import pallas as pl
from jax.experimental.pallas import tpu as pltpu
from jax.experimental.pallas import tpu_sc as plsc

E = 16
K = 2
DI = 1024
DH = 2048
DO = 1024
N = 8192

GT = 256
FT = 512
NT = N * K // FT + E
M_PAD = NT * FT


def _gating_body(x_ref, wg_ref, meta_ref, cnt_ref, loss_ref, base_ref, imp_ref):
    i = pl.program_id(0)

    @pl.when(i == 0)
    def _init():
        base_ref[...] = jnp.zeros_like(base_ref)
        imp_ref[...] = jnp.zeros_like(imp_ref)

    lg = jnp.dot(x_ref[...], wg_ref[...], preferred_element_type=jnp.float32)
    col = jax.lax.broadcasted_iota(jnp.int32, (GT, E), 1)
    m0 = jnp.max(lg, axis=1, keepdims=True)
    i0 = jnp.min(jnp.where(lg == m0, col, E), axis=1, keepdims=True)
    is0 = col == i0
    lg2 = jnp.where(is0, -jnp.inf, lg)
    m1 = jnp.max(lg2, axis=1, keepdims=True)
    i1 = jnp.min(jnp.where(lg2 == m1, col, E), axis=1, keepdims=True)
    is1 = col == i1
    a = jnp.exp(m1 - m0)
    g0 = 1.0 / (1.0 + a)
    g1 = a / (1.0 + a)
    P0 = is0.astype(jnp.float32)
    P1 = is1.astype(jnp.float32)

    rowi = jax.lax.broadcasted_iota(jnp.int32, (GT, GT), 0)
    colj = jax.lax.broadcasted_iota(jnp.int32, (GT, GT), 1)
    tri = (colj < rowi).astype(jnp.float32)
    cum0 = jnp.dot(tri, P0, preferred_element_type=jnp.float32)
    cum1 = jnp.dot(tri, P1, preferred_element_type=jnp.float32)
    c0 = jnp.sum(P0, axis=0, keepdims=True)
    base = base_ref[...]
    r0 = jnp.sum((cum0 + base) * P0, axis=1, keepdims=True)
    r1 = jnp.sum((cum1 + base + c0) * P1, axis=1, keepdims=True)

    cnt_tile = c0 + jnp.sum(P1, axis=0, keepdims=True)
    imp_tile = jnp.sum(g0 * P0 + g1 * P1, axis=0, keepdims=True)
    base_ref[...] = base + cnt_tile
    imp_ref[...] = imp_ref[...] + imp_tile

    meta_ref[...] = jnp.concatenate(
        [i0.astype(jnp.float32), i1.astype(jnp.float32), r0, r1, g0, g1,
         jnp.zeros((GT, 2), jnp.float32)], axis=1)

    @pl.when(i == pl.num_programs(0) - 1)
    def _fin():
        cnt = base_ref[...]
        imp = imp_ref[...]
        cnt_ref[...] = cnt

        def cv2(v):
            mean = jnp.sum(v) / E
            var = jnp.sum((v - mean) ** 2) / (E - 1)
            return var / (mean * mean + 1e-10)

        loss_ref[...] = (0.01 * (cv2(imp) + cv2(cnt))).reshape(1, 1)


def _gating(x, w_gate):
    return pl.pallas_call(
        _gating_body,
        grid=(N // GT,),
        in_specs=[
            pl.BlockSpec((GT, DI), lambda i: (i, 0)),
            pl.BlockSpec((DI, E), lambda i: (0, 0)),
        ],
        out_specs=[
            pl.BlockSpec((GT, 8), lambda i: (i, 0)),
            pl.BlockSpec((1, E), lambda i: (0, 0)),
            pl.BlockSpec((1, 1), lambda i: (0, 0)),
        ],
        out_shape=[
            jax.ShapeDtypeStruct((N, 8), jnp.float32),
            jax.ShapeDtypeStruct((1, E), jnp.float32),
            jax.ShapeDtypeStruct((1, 1), jnp.float32),
        ],
        scratch_shapes=[
            pltpu.VMEM((1, E), jnp.float32),
            pltpu.VMEM((1, E), jnp.float32),
        ],
    )(x, w_gate)


def _ffn_body(t2e_ref, xs_ref, w1_ref, b1_ref, w2_ref, b2_ref, sg_ref, o_ref):
    i = pl.program_id(0)

    @pl.when(t2e_ref[i] < E)
    def _go():
        h = jnp.dot(xs_ref[...].astype(jnp.bfloat16),
                    w1_ref[0].astype(jnp.bfloat16),
                    preferred_element_type=jnp.float32)
        h = jnp.maximum(h + b1_ref[0], 0.0)
        lg = jnp.dot(h.astype(jnp.bfloat16),
                     w2_ref[0].astype(jnp.bfloat16),
                     preferred_element_type=jnp.float32)
        lg = lg + b2_ref[0]
        m = jnp.max(lg, axis=1, keepdims=True)
        p = jnp.exp(lg - m)
        g = sg_ref[0, 0, :].reshape(FT, 1)
        o_ref[...] = p * (g / jnp.sum(p, axis=1, keepdims=True))


def _ffn(xs, W1, b1, W2, b2, sg, t2e):
    def emap(i, t2e_ref):
        return (jnp.minimum(t2e_ref[i], E - 1), 0, 0)

    def rmap(i, t2e_ref):
        return (jnp.minimum(i, t2e_ref[NT] - 1), 0)

    def rmap3(i, t2e_ref):
        return (jnp.minimum(i, t2e_ref[NT] - 1), 0, 0)

    grid_spec = pltpu.PrefetchScalarGridSpec(
        num_scalar_prefetch=1,
        grid=(NT,),
        in_specs=[
            pl.BlockSpec((FT, DI), rmap),
            pl.BlockSpec((1, DI, DH), emap),
            pl.BlockSpec((1, 1, DH), emap),
            pl.BlockSpec((1, DH, DO), emap),
            pl.BlockSpec((1, 1, DO), emap),
            pl.BlockSpec((1, 1, FT), rmap3),
        ],
        out_specs=pl.BlockSpec((FT, DO), rmap),
    )
    return pl.pallas_call(
        _ffn_body,
        grid_spec=grid_spec,
        out_shape=jax.ShapeDtypeStruct((M_PAD, DO), jnp.float32),
    )(t2e, xs, W1, b1.reshape(E, 1, DH), W2, b2.reshape(E, 1, DO),
      sg.reshape(NT, 1, FT))


NW = 32
PAIRS = N * K
PPW = PAIRS // NW
CH = 32
NCH = PPW // CH
NCH2 = NCH // 2
TPW = N // NW
TCH = CH // K

_SC_MESH = plsc.VectorSubcoreMesh(core_axis_name="c", subcore_axis_name="s")


@functools.partial(
    pl.kernel,
    out_type=[jax.ShapeDtypeStruct((M_PAD, DI), jnp.float32),
              jax.ShapeDtypeStruct((M_PAD,), jnp.float32)],
    mesh=_SC_MESH,
    scratch_types=[
        pltpu.VMEM((NCH, CH), jnp.int32),
        pltpu.VMEM((2, CH), jnp.float32),
        pltpu.VMEM((CH, DI), jnp.float32),
        pltpu.VMEM((CH, DI), jnp.float32),
        pltpu.SemaphoreType.DMA,
        pltpu.SemaphoreType.DMA,
        pltpu.SemaphoreType.DMA,
        pltpu.SemaphoreType.DMA,
        pltpu.SemaphoreType.DMA,
        pltpu.SemaphoreType.DMA,
    ],
)
def _sc_dispatch(x_hbm, pos_in_hbm, g_hbm,
                 xs_hbm, sg_hbm,
                 pos_v, g_v, rows0, rows1,
                 gsem0, gsem1, ssem0, ssem1, tsem0, tsem1):
    wid = lax.axis_index("s") * 2 + lax.axis_index("c")
    base0 = wid * PPW
    tok0 = jnp.where(base0 >= N, base0 - N, base0)

    def stage(c, b):
        base = base0 + c * CH
        pltpu.sync_copy(pos_in_hbm.at[pl.ds(base, CH)], pos_v.at[c])
        pltpu.sync_copy(g_hbm.at[pl.ds(base, CH)], g_v.at[b])

    def fire_read(c, rows, sem):
        pltpu.async_copy(x_hbm.at[pl.ds(tok0 + c * CH, CH)], rows, sem)

    stage(0, 0)
    fire_read(0, rows0, gsem0)

    def body(c2, carry):
        c0 = 2 * c2
        c1 = c0 + 1

        @pl.when(c2 >= 1)
        def _wait_b1():
            pltpu.make_async_copy(rows1, xs_hbm.at[pos_v.at[c1 - 2]], ssem1).wait()
            pltpu.make_async_copy(g_v.at[1], sg_hbm.at[pos_v.at[c1 - 2]], tsem1).wait()

        stage(c1, 1)
        fire_read(c1, rows1, gsem1)

        pltpu.make_async_copy(x_hbm.at[pl.ds(tok0, CH)], rows0, gsem0).wait()
        pltpu.async_copy(rows0, xs_hbm.at[pos_v.at[c0]], ssem0)
        pltpu.async_copy(g_v.at[0], sg_hbm.at[pos_v.at[c0]], tsem0)

        pltpu.make_async_copy(rows0, xs_hbm.at[pos_v.at[c0]], ssem0).wait()
        pltpu.make_async_copy(g_v.at[0], sg_hbm.at[pos_v.at[c0]], tsem0).wait()
        cn = jnp.minimum(c0 + 2, NCH - 1)
        stage(cn, 0)
        fire_read(cn, rows0, gsem0)

        pltpu.make_async_copy(x_hbm.at[pl.ds(tok0, CH)], rows1, gsem1).wait()
        pltpu.async_copy(rows1, xs_hbm.at[pos_v.at[c1]], ssem1)
        pltpu.async_copy(g_v.at[1], sg_hbm.at[pos_v.at[c1]], tsem1)
        return carry

    lax.fori_loop(0, NCH2, body, 0)
    pltpu.make_async_copy(x_hbm.at[pl.ds(tok0, CH)], rows0, gsem0).wait()
    pltpu.make_async_copy(rows1, xs_hbm.at[pos_v.at[NCH - 1]], ssem1).wait()
    pltpu.make_async_copy(g_v.at[1], sg_hbm.at[pos_v.at[NCH - 1]], tsem1).wait()


CCH = 16
NCHC = TPW // CCH


@functools.partial(
    pl.kernel,
    out_type=jax.ShapeDtypeStruct((N, DO), jnp.float32),
    mesh=_SC_MESH,
    scratch_types=[
        pltpu.VMEM((2, CCH), jnp.int32),
        pltpu.VMEM((2, CCH), jnp.int32),
        pltpu.VMEM((CCH, DO), jnp.float32),
        pltpu.VMEM((CCH, DO), jnp.float32),
        pltpu.VMEM((CCH, DO), jnp.float32),
        pltpu.VMEM((CCH, DO), jnp.float32),
        pltpu.SemaphoreType.DMA,
        pltpu.SemaphoreType.DMA,
        pltpu.SemaphoreType.DMA,
        pltpu.SemaphoreType.DMA,
    ],
)
def _sc_combine(o_hbm, pos_hbm,
                y_hbm,
                pa_v, pb_v, a0, a1, b0, b1, sa0, sa1, sb0, sb1):
    wid = lax.axis_index("s") * 2 + lax.axis_index("c")
    tbase0 = wid * TPW

    def stage_fire(c, b, av, bv, sa, sb):
        base = tbase0 + c * CCH
        pltpu.sync_copy(pos_hbm.at[pl.ds(base, CCH)], pa_v.at[b])
        pltpu.sync_copy(pos_hbm.at[pl.ds(N + base, CCH)], pb_v.at[b])
        pltpu.async_copy(o_hbm.at[pa_v.at[b]], av, sa)
        pltpu.async_copy(o_hbm.at[pb_v.at[b]], bv, sb)

    def compute(c, b, av, bv, sa, sb):
        pltpu.make_async_copy(o_hbm.at[pa_v.at[b]], av, sa).wait()
        pltpu.make_async_copy(o_hbm.at[pb_v.at[b]], bv, sb).wait()

        def tok(t, inner):
            for j in range(DO // 16):
                av[t, pl.ds(j * 16, 16)] = (av[t, pl.ds(j * 16, 16)]
                                            + bv[t, pl.ds(j * 16, 16)])
            return inner

        lax.fori_loop(0, CCH, tok, 0)
        pltpu.sync_copy(av, y_hbm.at[pl.ds(tbase0 + c * CCH, CCH)])

    stage_fire(0, 0, a0, b0, sa0, sb0)

    def body(c2, carry):
        c0 = 2 * c2
        c1 = c0 + 1
        stage_fire(c1, 1, a1, b1, sa1, sb1)
        compute(c0, 0, a0, b0, sa0, sb0)
        stage_fire(jnp.minimum(c0 + 2, NCHC - 1), 0, a0, b0, sa0, sb0)
        compute(c1, 1, a1, b1, sa1, sb1)
        return carry

    lax.fori_loop(0, NCHC // 2, body, 0)
    pltpu.make_async_copy(o_hbm.at[pa_v.at[0]], a0, sa0).wait()
    pltpu.make_async_copy(o_hbm.at[pb_v.at[0]], b0, sb0).wait()


def kernel(x, w_gate, W1, b1, W2, b2):
    meta, cnt, loss = _gating(x, w_gate)
    e_ids = meta[:, :K].astype(jnp.int32)
    ranks = meta[:, K:2 * K].astype(jnp.int32)
    gates = meta[:, 2 * K:3 * K]
    counts = cnt[0].astype(jnp.int32)

    tile_cnt = (counts + FT - 1) // FT
    tile_off = jnp.cumsum(tile_cnt)
    p_off = jnp.concatenate([jnp.zeros((1,), jnp.int32),
                             tile_off[:-1]]) * FT
    tids = jnp.arange(NT, dtype=jnp.int32)
    t2e = jnp.sum((tids[:, None] >= tile_off[None, :]).astype(jnp.int32), axis=1)
    t2e = jnp.concatenate([t2e, tile_off[-1:]])

    pos = p_off[e_ids] + ranks
    posf = pos.T.reshape(-1).astype(jnp.int32)
    gf = gates.T.reshape(-1)
    xs, sg = _sc_dispatch(x, posf, gf)
    o = _ffn(xs, W1, b1, W2, b2, sg, t2e)
    y = _sc_combine(o, posf)
    return (y, loss[0, 0])
